# Initial kernel scaffold; baseline (speedup 1.0000x reference)
import jax
import jax.numpy as jnp
from jax.experimental import pallas as pl


def kernel(x, edge_index, batch, W0, b0, W1, b1, W2, b2, Wp1, bp1, Wp2, bp2):
    raise NotImplementedError("write your pallas kernel here")


# speedup vs baseline: 5.8614x; 5.8614x over previous
import functools

import jax
import jax.numpy as jnp
from jax import lax
from jax.experimental import pallas as pl
from jax.experimental.pallas import tpu as pltpu
from jax.experimental.pallas import tpu_sc as plsc

N = 10000
E = 320000
D = 128
NG = 64
NC = 2
NS = 16
NW = NC * NS
CHUNK = 128
NP = 10112
ROWS_PER_TILE = NP // NS
CHUNKS_PER_W = 80
EP = NW * CHUNKS_PER_W * CHUNK
ER = EP // CHUNK


def _agg_body(h, src2d, dst2d, zerosd, out, idxs, idxd, rows, sem, acc):
    c = lax.axis_index("c")
    s = lax.axis_index("s")
    w = s * NC + c
    r0 = s * ROWS_PER_TILE
    pltpu.sync_copy(zerosd.at[pl.ds(r0, ROWS_PER_TILE)],
                    acc.at[pl.ds(r0, ROWS_PER_TILE)])
    pltpu.sync_copy(src2d.at[pl.ds(w * CHUNKS_PER_W, CHUNKS_PER_W)], idxs)
    pltpu.sync_copy(dst2d.at[pl.ds(w * CHUNKS_PER_W, CHUNKS_PER_W)], idxd)
    plsc.subcore_barrier()

    def body(j, carry):
        pltpu.async_copy(h.at[idxs.at[j]], rows, sem).wait()
        pltpu.sync_copy(rows, acc.at[idxd.at[j]], add=True)
        return carry

    lax.fori_loop(0, CHUNKS_PER_W, body, 0)
    plsc.subcore_barrier()
    pltpu.sync_copy(acc.at[pl.ds(r0, ROWS_PER_TILE)],
                    out.at[c, pl.ds(r0, ROWS_PER_TILE)])


_agg_sc = pl.kernel(
    _agg_body,
    mesh=plsc.VectorSubcoreMesh(core_axis_name="c", subcore_axis_name="s"),
    out_type=jax.ShapeDtypeStruct((NC, NP, D), jnp.float32),
    scratch_types=[
        pltpu.VMEM((CHUNKS_PER_W, CHUNK), jnp.int32),
        pltpu.VMEM((CHUNKS_PER_W, CHUNK), jnp.int32),
        pltpu.VMEM((CHUNK, D), jnp.float32),
        pltpu.SemaphoreType.DMA,
        pltpu.VMEM_SHARED((NP, D), jnp.float32),
    ],
)


EB = 6400
GSTEPS = E // EB
NQ = NP // CHUNK + 1


def _hist_body(d_ref, out_ref):
    @pl.when(pl.program_id(0) == 0)
    def _():
        out_ref[...] = jnp.zeros_like(out_ref)

    d = d_ref[...]
    q = jnp.floor(d * (1.0 / CHUNK))
    r = d - q * CHUNK
    qi = lax.broadcasted_iota(jnp.int32, (1, NQ), 1).astype(jnp.float32)
    ri = lax.broadcasted_iota(jnp.int32, (1, CHUNK), 1).astype(jnp.float32)
    ohq = jnp.where(q == qi, 1.0, 0.0)
    ohr = jnp.where(r == ri, 1.0, 0.0)
    out_ref[...] += lax.dot_general(ohq, ohr, (((0,), (0,)), ((), ())),
                                    preferred_element_type=jnp.float32)


_hist_tc = pl.pallas_call(
    _hist_body,
    grid=(GSTEPS,),
    in_specs=[pl.BlockSpec((EB, 1), lambda i: (i, 0))],
    out_specs=pl.BlockSpec((NQ, CHUNK), lambda i: (0, 0)),
    out_shape=jax.ShapeDtypeStruct((NQ, CHUNK), jnp.float32),
)


def _pre_body(x_ref, w_ref, b_ref, deg_ref, h_ref, dinv_ref):
    dinv = lax.rsqrt(deg_ref[...] + 1.0)
    dinv_ref[...] = dinv
    h = jnp.dot(x_ref[...], w_ref[...], preferred_element_type=jnp.float32)
    h_ref[...] = (h + b_ref[...]) * dinv


def _mid_body(p_ref, h_ref, dinv_ref, w_ref, b_ref, out_ref):
    dinv = dinv_ref[...]
    xl = jnp.maximum((p_ref[0] + p_ref[1] + h_ref[...]) * dinv, 0.0)
    h = jnp.dot(xl, w_ref[...], preferred_element_type=jnp.float32)
    out_ref[...] = (h + b_ref[...]) * dinv


def _post_body(p_ref, h_ref, dinv_ref, batch_ref, wp1_ref, bp1_ref,
               wp2_ref, bp2_ref, out_ref):
    xl = jnp.maximum((p_ref[0] + p_ref[1] + h_ref[...]) * dinv_ref[...], 0.0)
    seg = lax.broadcasted_iota(jnp.int32, (NG, 1), 0).astype(jnp.float32)
    oht = jnp.where(batch_ref[...] == seg, 1.0, 0.0)
    pool = jnp.dot(oht, xl, preferred_element_type=jnp.float32)
    cnt = jnp.sum(oht, axis=1, keepdims=True)
    pool = pool / jnp.clip(cnt, 1.0, None)
    hid = jnp.maximum(
        jnp.dot(pool, wp1_ref[...], preferred_element_type=jnp.float32)
        + bp1_ref[...], 0.0)
    out_ref[...] = (jnp.dot(hid, wp2_ref[...],
                            preferred_element_type=jnp.float32)
                    + bp2_ref[...])


def _tc_call(body, out_shape):
    return pl.pallas_call(body, out_shape=out_shape)


_pre_tc = _tc_call(_pre_body, (jax.ShapeDtypeStruct((NP, D), jnp.float32),
                               jax.ShapeDtypeStruct((NP, 1), jnp.float32)))
_mid_tc = _tc_call(_mid_body, jax.ShapeDtypeStruct((NP, D), jnp.float32))
_post_tc = _tc_call(_post_body, jax.ShapeDtypeStruct((NG, 1), jnp.float32))


def kernel(x, edge_index, batch, W0, b0, W1, b1, W2, b2, Wp1, bp1, Wp2, bp2):
    src = edge_index[0].astype(jnp.int32)
    dst = edge_index[1].astype(jnp.int32)
    npad = EP - E
    src2d = jnp.concatenate(
        [src, jnp.zeros((npad,), jnp.int32)]).reshape(ER, CHUNK)
    dst2d = jnp.concatenate(
        [dst, jnp.full((npad,), NP - 1, jnp.int32)]).reshape(ER, CHUNK)

    zerosd = jnp.zeros((NP, D), jnp.float32)
    xp = jnp.pad(x, ((0, NP - N), (0, 0)))
    batchf = jnp.pad(batch.astype(jnp.float32), (0, NP - N),
                     constant_values=float(NG)).reshape(1, NP)

    hist = _hist_tc(dst.astype(jnp.float32).reshape(E, 1))
    deg = hist.reshape(NQ * CHUNK)[:NP].reshape(NP, 1)
    h0, dinv = _pre_tc(xp, W0, b0.reshape(1, D), deg)
    p0 = _agg_sc(h0, src2d, dst2d, zerosd)
    h1 = _mid_tc(p0, h0, dinv, W1, b1.reshape(1, D))
    p1 = _agg_sc(h1, src2d, dst2d, zerosd)
    h2 = _mid_tc(p1, h1, dinv, W2, b2.reshape(1, D))
    p2 = _agg_sc(h2, src2d, dst2d, zerosd)
    out = _post_tc(p2, h2, dinv, batchf, Wp1, bp1.reshape(1, D),
                   Wp2, bp2.reshape(1, 1))
    return out[:, 0]


# speedup vs baseline: 6.2241x; 1.0619x over previous
import functools

import jax
import jax.numpy as jnp
from jax import lax
from jax.experimental import pallas as pl
from jax.experimental.pallas import tpu as pltpu
from jax.experimental.pallas import tpu_sc as plsc

N = 10000
E = 320000
D = 128
NG = 64
NC = 2
NS = 16
NW = NC * NS
CHUNK = 128
NP = 10112
ROWS_PER_TILE = NP // NS
CHUNKS_PER_W = 80
EP = NW * CHUNKS_PER_W * CHUNK
ER = EP // CHUNK


def _agg_body(h, src2d, dst2d, zerosd, out, idxs, idxd, rows, sem, acc):
    c = lax.axis_index("c")
    s = lax.axis_index("s")
    w = s * NC + c
    r0 = s * ROWS_PER_TILE
    pltpu.sync_copy(zerosd.at[pl.ds(r0, ROWS_PER_TILE)],
                    acc.at[pl.ds(r0, ROWS_PER_TILE)])
    pltpu.sync_copy(src2d.at[pl.ds(w * CHUNKS_PER_W, CHUNKS_PER_W)], idxs)
    pltpu.sync_copy(dst2d.at[pl.ds(w * CHUNKS_PER_W, CHUNKS_PER_W)], idxd)
    plsc.subcore_barrier()

    def body(j, carry):
        pltpu.async_copy(h.at[idxs.at[j]], rows, sem).wait()
        pltpu.sync_copy(rows, acc.at[idxd.at[j]], add=True)
        return carry

    lax.fori_loop(0, CHUNKS_PER_W, body, 0)
    plsc.subcore_barrier()
    pltpu.sync_copy(acc.at[pl.ds(r0, ROWS_PER_TILE)],
                    out.at[c, pl.ds(r0, ROWS_PER_TILE)])


_agg_sc = pl.kernel(
    _agg_body,
    mesh=plsc.VectorSubcoreMesh(core_axis_name="c", subcore_axis_name="s"),
    out_type=jax.ShapeDtypeStruct((NC, NP, D), jnp.float32),
    scratch_types=[
        pltpu.VMEM((CHUNKS_PER_W, CHUNK), jnp.int32),
        pltpu.VMEM((CHUNKS_PER_W, CHUNK), jnp.int32),
        pltpu.VMEM((CHUNK, D), jnp.float32),
        pltpu.SemaphoreType.DMA,
        pltpu.VMEM_SHARED((NP, D), jnp.float32),
    ],
)


EB = 6400
GSTEPS = E // EB
NQ = NP // CHUNK + 1


def _hist_body(d_ref, out_ref):
    @pl.when(pl.program_id(0) == 0)
    def _():
        out_ref[...] = jnp.zeros_like(out_ref)

    d = d_ref[...]
    q = jnp.floor(d * (1.0 / CHUNK))
    r = d - q * CHUNK
    qi = lax.broadcasted_iota(jnp.int32, (1, NQ), 1).astype(jnp.float32)
    ri = lax.broadcasted_iota(jnp.int32, (1, CHUNK), 1).astype(jnp.float32)
    ohq = jnp.where(q == qi, 1.0, 0.0)
    ohr = jnp.where(r == ri, 1.0, 0.0)
    out_ref[...] += lax.dot_general(ohq, ohr, (((0,), (0,)), ((), ())),
                                    preferred_element_type=jnp.float32)


_hist_tc = pl.pallas_call(
    _hist_body,
    grid=(GSTEPS,),
    in_specs=[pl.BlockSpec((EB, 1), lambda i: (i, 0))],
    out_specs=pl.BlockSpec((NQ, CHUNK), lambda i: (0, 0)),
    out_shape=jax.ShapeDtypeStruct((NQ, CHUNK), jnp.float32),
)


def _pre_body(x_ref, w_ref, b_ref, deg_ref, h_ref, dinv_ref):
    dinv = lax.rsqrt(deg_ref[...] + 1.0)
    dinv_ref[...] = dinv
    h = jnp.dot(x_ref[...], w_ref[...], preferred_element_type=jnp.float32)
    h_ref[...] = (h + b_ref[...]) * dinv


def _mid_body(p_ref, h_ref, dinv_ref, w_ref, b_ref, out_ref):
    dinv = dinv_ref[...]
    xl = jnp.maximum((p_ref[0] + p_ref[1] + h_ref[...]) * dinv, 0.0)
    h = jnp.dot(xl, w_ref[...], preferred_element_type=jnp.float32)
    out_ref[...] = (h + b_ref[...]) * dinv


def _post_body(p_ref, h_ref, dinv_ref, batch_ref, wp1_ref, bp1_ref,
               wp2_ref, bp2_ref, out_ref):
    xl = jnp.maximum((p_ref[0] + p_ref[1] + h_ref[...]) * dinv_ref[...], 0.0)
    seg = lax.broadcasted_iota(jnp.int32, (NG, 1), 0).astype(jnp.float32)
    oht = jnp.where(batch_ref[...] == seg, 1.0, 0.0)
    pool = jnp.dot(oht, xl, preferred_element_type=jnp.float32)
    cnt = jnp.sum(oht, axis=1, keepdims=True)
    pool = pool / jnp.clip(cnt, 1.0, None)
    hid = jnp.maximum(
        jnp.dot(pool, wp1_ref[...], preferred_element_type=jnp.float32)
        + bp1_ref[...], 0.0)
    out_ref[...] = (jnp.dot(hid, wp2_ref[...],
                            preferred_element_type=jnp.float32)
                    + bp2_ref[...])


def _tc_call(body, out_shape):
    return pl.pallas_call(body, out_shape=out_shape)


_pre_tc = _tc_call(_pre_body, (jax.ShapeDtypeStruct((NP, D), jnp.float32),
                               jax.ShapeDtypeStruct((NP, 1), jnp.float32)))
_mid_tc = _tc_call(_mid_body, jax.ShapeDtypeStruct((NP, D), jnp.float32))
_post_tc = _tc_call(_post_body, jax.ShapeDtypeStruct((NG, 1), jnp.float32))


def kernel(x, edge_index, batch, W0, b0, W1, b1, W2, b2, Wp1, bp1, Wp2, bp2):
    src = edge_index[0].astype(jnp.int32)
    dst = edge_index[1].astype(jnp.int32)
    npad = EP - E
    trash = N + jnp.arange(npad, dtype=jnp.int32) % (NP - N)
    src2d = jnp.concatenate(
        [src, jnp.zeros((npad,), jnp.int32)]).reshape(ER, CHUNK)
    dst2d = jnp.concatenate([dst, trash]).reshape(ER, CHUNK)

    zerosd = jnp.zeros((NP, D), jnp.float32)
    xp = jnp.pad(x, ((0, NP - N), (0, 0)))
    batchf = jnp.pad(batch.astype(jnp.float32), (0, NP - N),
                     constant_values=float(NG)).reshape(1, NP)

    hist = _hist_tc(dst.astype(jnp.float32).reshape(E, 1))
    deg = hist.reshape(NQ * CHUNK)[:NP].reshape(NP, 1)
    h0, dinv = _pre_tc(xp, W0, b0.reshape(1, D), deg)
    p0 = _agg_sc(h0, src2d, dst2d, zerosd)
    h1 = _mid_tc(p0, h0, dinv, W1, b1.reshape(1, D))
    p1 = _agg_sc(h1, src2d, dst2d, zerosd)
    h2 = _mid_tc(p1, h1, dinv, W2, b2.reshape(1, D))
    p2 = _agg_sc(h2, src2d, dst2d, zerosd)
    out = _post_tc(p2, h2, dinv, batchf, Wp1, bp1.reshape(1, D),
                   Wp2, bp2.reshape(1, 1))
    return out[:, 0]


# speedup vs baseline: 7.2393x; 1.1631x over previous
import functools

import jax
import jax.numpy as jnp
from jax import lax
from jax.experimental import pallas as pl
from jax.experimental.pallas import tpu as pltpu
from jax.experimental.pallas import tpu_sc as plsc

N = 10000
E = 320000
D = 128
NG = 64
NC = 2
NS = 16
NW = NC * NS
CHUNK = 128
NP = 10112
ROWS_PER_TILE = NP // NS
CHUNKS_PER_W = 80
EP = NW * CHUNKS_PER_W * CHUNK
ER = EP // CHUNK


CH0 = 120
CH1 = 40
ROWS0 = NS * CH0


def _agg_body(h, src2d, dst2d, zerosd, out, idxs, idxd, rows, sem, acc):
    c = lax.axis_index("c")
    s = lax.axis_index("s")
    r0 = s * ROWS_PER_TILE
    pltpu.sync_copy(zerosd.at[pl.ds(r0, ROWS_PER_TILE)],
                    acc.at[pl.ds(r0, ROWS_PER_TILE)])

    @pl.when(c == 0)
    def _():
        pltpu.sync_copy(src2d.at[pl.ds(s * CH0, CH0)], idxs.at[pl.ds(0, CH0)])
        pltpu.sync_copy(dst2d.at[pl.ds(s * CH0, CH0)], idxd.at[pl.ds(0, CH0)])

    @pl.when(c == 1)
    def _():
        pltpu.sync_copy(src2d.at[pl.ds(ROWS0 + s * CH1, CH1)],
                        idxs.at[pl.ds(0, CH1)])
        pltpu.sync_copy(dst2d.at[pl.ds(ROWS0 + s * CH1, CH1)],
                        idxd.at[pl.ds(0, CH1)])

    nch = jnp.where(c == 0, CH0, CH1)
    plsc.subcore_barrier()

    def body(j, carry):
        pltpu.async_copy(h.at[idxs.at[j]], rows, sem).wait()
        pltpu.sync_copy(rows, acc.at[idxd.at[j]], add=True)
        return carry

    lax.fori_loop(0, nch, body, 0)
    plsc.subcore_barrier()
    pltpu.sync_copy(acc.at[pl.ds(r0, ROWS_PER_TILE)],
                    out.at[c, pl.ds(r0, ROWS_PER_TILE)])


_agg_sc = pl.kernel(
    _agg_body,
    mesh=plsc.VectorSubcoreMesh(core_axis_name="c", subcore_axis_name="s"),
    out_type=jax.ShapeDtypeStruct((NC, NP, D), jnp.float32),
    scratch_types=[
        pltpu.VMEM((CH0, CHUNK), jnp.int32),
        pltpu.VMEM((CH0, CHUNK), jnp.int32),
        pltpu.VMEM((CHUNK, D), jnp.float32),
        pltpu.SemaphoreType.DMA,
        pltpu.VMEM_SHARED((NP, D), jnp.float32),
    ],
)


EB = 6400
GSTEPS = E // EB
NQ = NP // CHUNK + 1


def _hist_body(d_ref, out_ref):
    @pl.when(pl.program_id(0) == 0)
    def _():
        out_ref[...] = jnp.zeros_like(out_ref)

    d = d_ref[...]
    q = jnp.floor(d * (1.0 / CHUNK))
    r = d - q * CHUNK
    qi = lax.broadcasted_iota(jnp.int32, (1, NQ), 1).astype(jnp.float32)
    ri = lax.broadcasted_iota(jnp.int32, (1, CHUNK), 1).astype(jnp.float32)
    ohq = jnp.where(q == qi, 1.0, 0.0)
    ohr = jnp.where(r == ri, 1.0, 0.0)
    out_ref[...] += lax.dot_general(ohq, ohr, (((0,), (0,)), ((), ())),
                                    preferred_element_type=jnp.float32)


_hist_tc = pl.pallas_call(
    _hist_body,
    grid=(GSTEPS,),
    in_specs=[pl.BlockSpec((EB, 1), lambda i: (i, 0))],
    out_specs=pl.BlockSpec((NQ, CHUNK), lambda i: (0, 0)),
    out_shape=jax.ShapeDtypeStruct((NQ, CHUNK), jnp.float32),
)


def _pre_body(x_ref, w_ref, b_ref, deg_ref, h_ref, dinv_ref):
    dinv = lax.rsqrt(deg_ref[...] + 1.0)
    dinv_ref[...] = dinv
    h = jnp.dot(x_ref[...], w_ref[...], preferred_element_type=jnp.float32)
    h_ref[...] = (h + b_ref[...]) * dinv


def _mid_body(p_ref, h_ref, dinv_ref, w_ref, b_ref, out_ref):
    dinv = dinv_ref[...]
    xl = jnp.maximum((p_ref[0] + p_ref[1] + h_ref[...]) * dinv, 0.0)
    h = jnp.dot(xl, w_ref[...], preferred_element_type=jnp.float32)
    out_ref[...] = (h + b_ref[...]) * dinv


def _post_body(p_ref, h_ref, dinv_ref, batch_ref, wp1_ref, bp1_ref,
               wp2_ref, bp2_ref, out_ref):
    xl = jnp.maximum((p_ref[0] + p_ref[1] + h_ref[...]) * dinv_ref[...], 0.0)
    seg = lax.broadcasted_iota(jnp.int32, (NG, 1), 0).astype(jnp.float32)
    oht = jnp.where(batch_ref[...] == seg, 1.0, 0.0)
    pool = jnp.dot(oht, xl, preferred_element_type=jnp.float32)
    cnt = jnp.sum(oht, axis=1, keepdims=True)
    pool = pool / jnp.clip(cnt, 1.0, None)
    hid = jnp.maximum(
        jnp.dot(pool, wp1_ref[...], preferred_element_type=jnp.float32)
        + bp1_ref[...], 0.0)
    out_ref[...] = (jnp.dot(hid, wp2_ref[...],
                            preferred_element_type=jnp.float32)
                    + bp2_ref[...])


def _tc_call(body, out_shape):
    return pl.pallas_call(body, out_shape=out_shape)


_pre_tc = _tc_call(_pre_body, (jax.ShapeDtypeStruct((NP, D), jnp.float32),
                               jax.ShapeDtypeStruct((NP, 1), jnp.float32)))
_mid_tc = _tc_call(_mid_body, jax.ShapeDtypeStruct((NP, D), jnp.float32))
_post_tc = _tc_call(_post_body, jax.ShapeDtypeStruct((NG, 1), jnp.float32))


def kernel(x, edge_index, batch, W0, b0, W1, b1, W2, b2, Wp1, bp1, Wp2, bp2):
    src = edge_index[0].astype(jnp.int32)
    dst = edge_index[1].astype(jnp.int32)
    npad = EP - E
    trash = N + jnp.arange(npad, dtype=jnp.int32) % (NP - N)
    src2d = jnp.concatenate(
        [src, jnp.zeros((npad,), jnp.int32)]).reshape(ER, CHUNK)
    dst2d = jnp.concatenate([dst, trash]).reshape(ER, CHUNK)

    zerosd = jnp.zeros((NP, D), jnp.float32)
    xp = jnp.pad(x, ((0, NP - N), (0, 0)))
    batchf = jnp.pad(batch.astype(jnp.float32), (0, NP - N),
                     constant_values=float(NG)).reshape(1, NP)

    hist = _hist_tc(dst.astype(jnp.float32).reshape(E, 1))
    deg = hist.reshape(NQ * CHUNK)[:NP].reshape(NP, 1)
    h0, dinv = _pre_tc(xp, W0, b0.reshape(1, D), deg)
    p0 = _agg_sc(h0, src2d, dst2d, zerosd)
    h1 = _mid_tc(p0, h0, dinv, W1, b1.reshape(1, D))
    p1 = _agg_sc(h1, src2d, dst2d, zerosd)
    h2 = _mid_tc(p1, h1, dinv, W2, b2.reshape(1, D))
    p2 = _agg_sc(h2, src2d, dst2d, zerosd)
    out = _post_tc(p2, h2, dinv, batchf, Wp1, bp1.reshape(1, D),
                   Wp2, bp2.reshape(1, 1))
    return out[:, 0]
